# baseline (speedup 1.0000x reference)
import jax
import jax.numpy as jnp
from jax.experimental import pallas as pl

_HI = jax.lax.Precision.HIGHEST


def _bank_kernel(x_ref, y_ref, hq_ref, b2_ref, keys_ref, ym_ref, xfeat_ref):
    x = x_ref[...]
    hq = hq_ref[...]
    pieces = [jnp.dot(x[:, b, :], hq, preferred_element_type=jnp.float32,
                      precision=_HI) for b in range(32)]
    x_feat = jnp.concatenate(pieces, axis=1)
    xfeat_ref[...] = x_feat
    keys_un = jnp.dot(x_feat, b2_ref[...],
                      preferred_element_type=jnp.float32, precision=_HI)
    ss = jnp.sum(keys_un * keys_un, axis=-1, keepdims=True)
    nrm = jnp.maximum(jnp.sqrt(ss), 1e-12)
    keys_ref[...] = keys_un / nrm
    y = y_ref[...]
    for j in range(6):
        ym_ref[:, j * 16:(j + 1) * 16] = jnp.dot(
            y[:, j, :], hq, preferred_element_type=jnp.float32, precision=_HI)


def kernel(all_x, all_y, w_ext, b_ext, w_cp, b_cp, W_enc):
    M, T, N = all_x.shape
    P = all_y.shape[1]
    BINS, D = W_enc.shape

    x3 = all_x.reshape(M, BINS, (T // BINS) * N)
    y3 = all_y.reshape(M, (P * N) // 128, 128)

    lane = jnp.arange(128)
    grp = jnp.arange(16)
    Hq = jnp.where(lane[:, None] // N == grp[None, :], 1.0 / N, 0.0
                   ).astype(jnp.float32)
    B2 = jnp.repeat(W_enc, T // BINS, axis=0) / (T // BINS)

    BM = 256
    grid = (M // BM,)
    keys, ym, x_feat = pl.pallas_call(
        _bank_kernel,
        grid=grid,
        in_specs=[
            pl.BlockSpec((BM, BINS, 128), lambda i: (i, 0, 0)),
            pl.BlockSpec((BM, 6, 128), lambda i: (i, 0, 0)),
            pl.BlockSpec((128, 16), lambda i: (0, 0)),
            pl.BlockSpec((T, D), lambda i: (0, 0)),
        ],
        out_specs=[
            pl.BlockSpec((BM, D), lambda i: (i, 0)),
            pl.BlockSpec((BM, P), lambda i: (i, 0)),
            pl.BlockSpec((BM, T), lambda i: (i, 0)),
        ],
        out_shape=[
            jax.ShapeDtypeStruct((M, D), jnp.float32),
            jax.ShapeDtypeStruct((M, P), jnp.float32),
            jax.ShapeDtypeStruct((M, T), jnp.float32),
        ],
    )(x3, y3, Hq, B2)

    extreme_probs = jax.nn.sigmoid(x_feat @ w_ext + b_ext)
    near_end_scores = jax.nn.sigmoid(x_feat[:, -64:] @ w_cp + b_cp)
    labels = jnp.zeros((M,), dtype=jnp.int32)
    labels = jnp.where(extreme_probs > 0.5, jnp.int32(1), labels)
    labels = jnp.where(near_end_scores > 0.5, jnp.int32(2), labels)
    return (keys, keys, keys, ym, labels)


# speedup vs baseline: 4.8051x; 4.8051x over previous
import jax
import jax.numpy as jnp
from jax.experimental import pallas as pl

_HI = jax.lax.Precision.HIGHEST


def _bank_kernel(x_ref, y_ref, b2_ref, keys_ref, ym_ref, xfeat_ref):
    x = x_ref[...]
    x_feat = jnp.sum(x, axis=1) * 0.125
    xfeat_ref[...] = x_feat
    keys_un = jnp.dot(x_feat, b2_ref[...],
                      preferred_element_type=jnp.float32, precision=_HI)
    ss = jnp.sum(keys_un * keys_un, axis=-1, keepdims=True)
    nrm = jnp.maximum(jnp.sqrt(ss), 1e-12)
    keys_ref[...] = keys_un / nrm
    y = y_ref[...]
    ym_ref[...] = jnp.sum(y, axis=1) * 0.125


def kernel(all_x, all_y, w_ext, b_ext, w_cp, b_cp, W_enc):
    M, T, N = all_x.shape
    P = all_y.shape[1]
    BINS, D = W_enc.shape

    xt = jnp.transpose(all_x, (0, 2, 1))
    yt = jnp.transpose(all_y, (1, 2, 0))

    B2 = jnp.repeat(W_enc, T // BINS, axis=0) / (T // BINS)

    BM = 256
    grid = (M // BM,)
    keys, ym_t, x_feat = pl.pallas_call(
        _bank_kernel,
        grid=grid,
        in_specs=[
            pl.BlockSpec((BM, N, T), lambda i: (i, 0, 0)),
            pl.BlockSpec((P, N, BM), lambda i: (0, 0, i)),
            pl.BlockSpec((T, D), lambda i: (0, 0)),
        ],
        out_specs=[
            pl.BlockSpec((BM, D), lambda i: (i, 0)),
            pl.BlockSpec((P, BM), lambda i: (0, i)),
            pl.BlockSpec((BM, T), lambda i: (i, 0)),
        ],
        out_shape=[
            jax.ShapeDtypeStruct((M, D), jnp.float32),
            jax.ShapeDtypeStruct((P, M), jnp.float32),
            jax.ShapeDtypeStruct((M, T), jnp.float32),
        ],
    )(xt, yt, B2)
    ym = ym_t.T

    extreme_probs = jax.nn.sigmoid(x_feat @ w_ext + b_ext)
    near_end_scores = jax.nn.sigmoid(x_feat[:, -64:] @ w_cp + b_cp)
    labels = jnp.zeros((M,), dtype=jnp.int32)
    labels = jnp.where(extreme_probs > 0.5, jnp.int32(1), labels)
    labels = jnp.where(near_end_scores > 0.5, jnp.int32(2), labels)
    return (keys, keys, keys, ym, labels)
